# baseline (speedup 1.0000x reference)
import functools

import jax
import jax.numpy as jnp
from jax import lax
from jax.experimental import pallas as pl
from jax.experimental.pallas import tpu as pltpu
from jax.experimental.pallas import tpu_sc as plsc

_B = 32
_C = 3
_H = 512
_W = 512
_BT = 22
_BS = _B - _BT
_NC = 2
_NS = 16
_NW = _NC * _NS
_ROWS = 32
_CHUNKS_PER_SLAB = _C * (_H // _ROWS)
_NCHUNK = _BS * _CHUNKS_PER_SLAB
_Q = _NCHUNK // _NW
assert _Q * _NW == _NCHUNK
_LANES = 16
_NBUF = 2
_VECS = _ROWS * _W // _LANES


def _mesh():
    return plsc.VectorSubcoreMesh(core_axis_name="c", subcore_axis_name="s")


@functools.partial(
    pl.kernel,
    mesh=_mesh(),
    out_type=[
        jax.ShapeDtypeStruct((_NW * _LANES,), jnp.float32),
        jax.ShapeDtypeStruct((_NW * _LANES,), jnp.int32),
    ],
    scratch_types=[
        pltpu.VMEM((_NBUF, _ROWS, _W), jnp.float32),
        pltpu.VMEM((_NBUF, _ROWS, _W), jnp.float32),
        pltpu.VMEM((_NBUF, _ROWS, _W), jnp.int32),
        pltpu.VMEM((_LANES,), jnp.float32),
        pltpu.VMEM((_LANES,), jnp.int32),
        pltpu.SemaphoreType.DMA((_NBUF,)),
    ],
)
def _masked_l1_sc(hat, obs, mask, out_s, out_c, h_v, o_v, m_v, acc_s_v, acc_c_v,
                  sems):
    wid = lax.axis_index("s") * _NC + lax.axis_index("c")
    g0 = wid * _Q

    def chunk_slices(local_idx):
        g = g0 + local_idx
        slab = lax.div(g, _CHUNKS_PER_SLAB)
        rem = lax.rem(g, _CHUNKS_PER_SLAB)
        b_idx = _BT + slab
        ch = lax.shift_right_logical(rem, 4)
        r0 = lax.mul(lax.bitwise_and(rem, 15), _ROWS)
        return b_idx, ch, r0

    def issue(b, local_idx):
        bi, ch, r0 = chunk_slices(local_idx)
        pltpu.async_copy(hat.at[bi, ch, pl.ds(r0, _ROWS), :], h_v.at[b],
                         sems.at[b])
        pltpu.async_copy(obs.at[bi, ch, pl.ds(r0, _ROWS), :], o_v.at[b],
                         sems.at[b])
        pltpu.async_copy(mask.at[bi, ch, pl.ds(r0, _ROWS), :], m_v.at[b],
                         sems.at[b])

    def drain(b, local_idx):
        bi, ch, r0 = chunk_slices(local_idx)
        pltpu.make_async_copy(hat.at[bi, ch, pl.ds(r0, _ROWS), :], h_v.at[b],
                              sems.at[b]).wait()
        pltpu.make_async_copy(obs.at[bi, ch, pl.ds(r0, _ROWS), :], o_v.at[b],
                              sems.at[b]).wait()
        pltpu.make_async_copy(mask.at[bi, ch, pl.ds(r0, _ROWS), :], m_v.at[b],
                              sems.at[b]).wait()

    issue(0, 0)
    issue(1, 1)

    def pair_body(i, carry):
        j = i * _NBUF

        def one(b, carry2):
            s, c = carry2
            jj = j + b
            drain(b, jj)

            def step(k, carry3):
                s3, c3 = carry3
                r = lax.shift_right_logical(k, 5)
                col = lax.mul(lax.bitwise_and(k, 31), _LANES)
                h = h_v[b, r, pl.ds(col, _LANES)]
                o = o_v[b, r, pl.ds(col, _LANES)]
                m = m_v[b, r, pl.ds(col, _LANES)]
                d = jnp.abs(h - o)
                s3 = s3 + d * m.astype(jnp.float32)
                c3 = c3 + m
                return s3, c3

            s, c = lax.fori_loop(0, _VECS, step, (s, c), unroll=8)

            @pl.when(jj + _NBUF < _Q)
            def _():
                issue(b, jj + _NBUF)

            return s, c

        for b in range(_NBUF):
            carry = one(b, carry)
        return carry

    s0 = jnp.zeros((_LANES,), jnp.float32)
    c0 = jnp.zeros((_LANES,), jnp.int32)
    s, c = lax.fori_loop(0, _Q // _NBUF, pair_body, (s0, c0))

    acc_s_v[...] = s
    acc_c_v[...] = c
    pltpu.sync_copy(acc_s_v, out_s.at[pl.ds(wid * _LANES, _LANES)])
    pltpu.sync_copy(acc_c_v, out_c.at[pl.ds(wid * _LANES, _LANES)])


def _tc_body(h_hbm, o_hbm, m_hbm, out_s_ref, out_c_ref,
             h_v, o_v, m_v, acc_s, acc_c, sems):
    def issue(b, bi):
        for ch in range(_C):
            pltpu.make_async_copy(h_hbm.at[bi, ch], h_v.at[b, ch],
                                  sems.at[b]).start()
            pltpu.make_async_copy(o_hbm.at[bi, ch], o_v.at[b, ch],
                                  sems.at[b]).start()
            pltpu.make_async_copy(m_hbm.at[bi, ch], m_v.at[b, ch],
                                  sems.at[b]).start()

    def drain(b, bi):
        for ch in range(_C):
            pltpu.make_async_copy(h_hbm.at[bi, ch], h_v.at[b, ch],
                                  sems.at[b]).wait()
            pltpu.make_async_copy(o_hbm.at[bi, ch], o_v.at[b, ch],
                                  sems.at[b]).wait()
            pltpu.make_async_copy(m_hbm.at[bi, ch], m_v.at[b, ch],
                                  sems.at[b]).wait()

    acc_s[...] = jnp.zeros((_H, _W), jnp.float32)
    acc_c[...] = jnp.zeros((_H, _W), jnp.int32)

    issue(0, 0)
    issue(1, 1)

    def pair_body(i, _):
        j = i * _NBUF

        def one(b):
            bi = j + b
            drain(b, bi)
            for ch in range(_C):
                h = h_v[b, ch]
                o = o_v[b, ch]
                m = m_v[b, ch]
                d = jnp.abs(h - o)
                acc_s[...] += d * m.astype(jnp.float32)
                acc_c[...] += m

            @pl.when(bi + _NBUF < _BT)
            def _():
                issue(b, bi + _NBUF)

        for b in range(_NBUF):
            one(b)
        return 0

    lax.fori_loop(0, _BT // _NBUF, pair_body, 0)

    out_s_ref[0] = jnp.sum(acc_s[...])
    out_c_ref[0] = jnp.sum(acc_c[...])


_tc_part = pl.pallas_call(
    _tc_body,
    in_specs=[
        pl.BlockSpec(memory_space=pltpu.HBM),
        pl.BlockSpec(memory_space=pltpu.HBM),
        pl.BlockSpec(memory_space=pltpu.HBM),
    ],
    out_specs=[
        pl.BlockSpec(memory_space=pltpu.SMEM),
        pl.BlockSpec(memory_space=pltpu.SMEM),
    ],
    out_shape=[
        jax.ShapeDtypeStruct((1,), jnp.float32),
        jax.ShapeDtypeStruct((1,), jnp.int32),
    ],
    scratch_shapes=[
        pltpu.VMEM((_NBUF, _C, _H, _W), jnp.float32),
        pltpu.VMEM((_NBUF, _C, _H, _W), jnp.float32),
        pltpu.VMEM((_NBUF, _C, _H, _W), jnp.int32),
        pltpu.VMEM((_H, _W), jnp.float32),
        pltpu.VMEM((_H, _W), jnp.int32),
        pltpu.SemaphoreType.DMA((_NBUF,)),
    ],
)


@jax.jit
def kernel(hat, obs, mask):
    part_s, part_c = _masked_l1_sc(hat, obs, mask)
    tc_s, tc_c = _tc_part(hat, obs, mask)
    total_s = jnp.sum(part_s) + tc_s[0]
    total_c = jnp.sum(part_c) + tc_c[0]
    return total_s / total_c.astype(jnp.float32)


# speedup vs baseline: 1.0007x; 1.0007x over previous
import functools

import jax
import jax.numpy as jnp
from jax import lax
from jax.experimental import pallas as pl
from jax.experimental.pallas import tpu as pltpu
from jax.experimental.pallas import tpu_sc as plsc

_B = 32
_C = 3
_H = 512
_W = 512
_BT = 20
_BS = _B - _BT
_NC = 2
_NS = 16
_NW = _NC * _NS
_ROWS = 32
_CHUNKS_PER_SLAB = _C * (_H // _ROWS)
_NCHUNK = _BS * _CHUNKS_PER_SLAB
_Q = _NCHUNK // _NW
assert _Q * _NW == _NCHUNK
_LANES = 16
_NBUF = 2
_VECS = _ROWS * _W // _LANES
_TCBUF = 4


def _mesh():
    return plsc.VectorSubcoreMesh(core_axis_name="c", subcore_axis_name="s")


@functools.partial(
    pl.kernel,
    mesh=_mesh(),
    out_type=[
        jax.ShapeDtypeStruct((_NW * _LANES,), jnp.float32),
        jax.ShapeDtypeStruct((_NW * _LANES,), jnp.int32),
    ],
    scratch_types=[
        pltpu.VMEM((_NBUF, _ROWS, _W), jnp.float32),
        pltpu.VMEM((_NBUF, _ROWS, _W), jnp.float32),
        pltpu.VMEM((_NBUF, _ROWS, _W), jnp.int32),
        pltpu.VMEM((_LANES,), jnp.float32),
        pltpu.VMEM((_LANES,), jnp.int32),
        pltpu.SemaphoreType.DMA((_NBUF,)),
    ],
)
def _masked_l1_sc(hat, obs, mask, out_s, out_c, h_v, o_v, m_v, acc_s_v, acc_c_v,
                  sems):
    wid = lax.axis_index("s") * _NC + lax.axis_index("c")
    g0 = wid * _Q

    def chunk_slices(local_idx):
        g = g0 + local_idx
        slab = lax.div(g, _CHUNKS_PER_SLAB)
        rem = lax.rem(g, _CHUNKS_PER_SLAB)
        b_idx = _BT + slab
        ch = lax.shift_right_logical(rem, 4)
        r0 = lax.mul(lax.bitwise_and(rem, 15), _ROWS)
        return b_idx, ch, r0

    def issue(b, local_idx):
        bi, ch, r0 = chunk_slices(local_idx)
        pltpu.async_copy(hat.at[bi, ch, pl.ds(r0, _ROWS), :], h_v.at[b],
                         sems.at[b])
        pltpu.async_copy(obs.at[bi, ch, pl.ds(r0, _ROWS), :], o_v.at[b],
                         sems.at[b])
        pltpu.async_copy(mask.at[bi, ch, pl.ds(r0, _ROWS), :], m_v.at[b],
                         sems.at[b])

    def drain(b, local_idx):
        bi, ch, r0 = chunk_slices(local_idx)
        pltpu.make_async_copy(hat.at[bi, ch, pl.ds(r0, _ROWS), :], h_v.at[b],
                              sems.at[b]).wait()
        pltpu.make_async_copy(obs.at[bi, ch, pl.ds(r0, _ROWS), :], o_v.at[b],
                              sems.at[b]).wait()
        pltpu.make_async_copy(mask.at[bi, ch, pl.ds(r0, _ROWS), :], m_v.at[b],
                              sems.at[b]).wait()

    issue(0, 0)
    issue(1, 1)

    def pair_body(i, carry):
        j = i * _NBUF

        def one(b, carry2):
            s, c = carry2
            jj = j + b
            drain(b, jj)

            def step(k, carry3):
                s3, c3 = carry3
                r = lax.shift_right_logical(k, 5)
                col = lax.mul(lax.bitwise_and(k, 31), _LANES)
                h = h_v[b, r, pl.ds(col, _LANES)]
                o = o_v[b, r, pl.ds(col, _LANES)]
                m = m_v[b, r, pl.ds(col, _LANES)]
                d = jnp.abs(h - o)
                s3 = s3 + d * m.astype(jnp.float32)
                c3 = c3 + m
                return s3, c3

            s, c = lax.fori_loop(0, _VECS, step, (s, c), unroll=8)

            @pl.when(jj + _NBUF < _Q)
            def _():
                issue(b, jj + _NBUF)

            return s, c

        for b in range(_NBUF):
            carry = one(b, carry)
        return carry

    s0 = jnp.zeros((_LANES,), jnp.float32)
    c0 = jnp.zeros((_LANES,), jnp.int32)
    s, c = lax.fori_loop(0, _Q // _NBUF, pair_body, (s0, c0))

    acc_s_v[...] = s
    acc_c_v[...] = c
    pltpu.sync_copy(acc_s_v, out_s.at[pl.ds(wid * _LANES, _LANES)])
    pltpu.sync_copy(acc_c_v, out_c.at[pl.ds(wid * _LANES, _LANES)])


def _tc_body(h_hbm, o_hbm, m_hbm, out_s_ref, out_c_ref,
             h_v, o_v, m_v, acc_s, acc_c, sems):
    def issue(b, bi):
        for ch in range(_C):
            pltpu.make_async_copy(h_hbm.at[bi, ch], h_v.at[b, ch],
                                  sems.at[b]).start()
            pltpu.make_async_copy(o_hbm.at[bi, ch], o_v.at[b, ch],
                                  sems.at[b]).start()
            pltpu.make_async_copy(m_hbm.at[bi, ch], m_v.at[b, ch],
                                  sems.at[b]).start()

    def drain(b, bi):
        for ch in range(_C):
            pltpu.make_async_copy(h_hbm.at[bi, ch], h_v.at[b, ch],
                                  sems.at[b]).wait()
            pltpu.make_async_copy(o_hbm.at[bi, ch], o_v.at[b, ch],
                                  sems.at[b]).wait()
            pltpu.make_async_copy(m_hbm.at[bi, ch], m_v.at[b, ch],
                                  sems.at[b]).wait()

    acc_s[...] = jnp.zeros((_H, _W), jnp.float32)
    acc_c[...] = jnp.zeros((_H, _W), jnp.int32)

    for b0 in range(_TCBUF):
        issue(b0, b0)

    def ring_body(i, _):
        j = i * _TCBUF

        def one(b):
            bi = j + b
            drain(b, bi)
            for ch in range(_C):
                h = h_v[b, ch]
                o = o_v[b, ch]
                m = m_v[b, ch]
                d = jnp.abs(h - o)
                acc_s[...] += d * m.astype(jnp.float32)
                acc_c[...] += m

            @pl.when(bi + _TCBUF < _BT)
            def _():
                issue(b, bi + _TCBUF)

        for b in range(_TCBUF):
            one(b)
        return 0

    lax.fori_loop(0, _BT // _TCBUF, ring_body, 0)

    out_s_ref[0] = jnp.sum(acc_s[...])
    out_c_ref[0] = jnp.sum(acc_c[...])


_tc_part = pl.pallas_call(
    _tc_body,
    in_specs=[
        pl.BlockSpec(memory_space=pltpu.HBM),
        pl.BlockSpec(memory_space=pltpu.HBM),
        pl.BlockSpec(memory_space=pltpu.HBM),
    ],
    out_specs=[
        pl.BlockSpec(memory_space=pltpu.SMEM),
        pl.BlockSpec(memory_space=pltpu.SMEM),
    ],
    out_shape=[
        jax.ShapeDtypeStruct((1,), jnp.float32),
        jax.ShapeDtypeStruct((1,), jnp.int32),
    ],
    scratch_shapes=[
        pltpu.VMEM((_TCBUF, _C, _H, _W), jnp.float32),
        pltpu.VMEM((_TCBUF, _C, _H, _W), jnp.float32),
        pltpu.VMEM((_TCBUF, _C, _H, _W), jnp.int32),
        pltpu.VMEM((_H, _W), jnp.float32),
        pltpu.VMEM((_H, _W), jnp.int32),
        pltpu.SemaphoreType.DMA((_TCBUF,)),
    ],
)


@jax.jit
def kernel(hat, obs, mask):
    part_s, part_c = _masked_l1_sc(hat, obs, mask)
    tc_s, tc_c = _tc_part(hat, obs, mask)
    total_s = jnp.sum(part_s) + tc_s[0]
    total_c = jnp.sum(part_c) + tc_c[0]
    return total_s / total_c.astype(jnp.float32)


# speedup vs baseline: 1.2266x; 1.2257x over previous
import functools

import jax
import jax.numpy as jnp
from jax import lax
from jax.experimental import pallas as pl
from jax.experimental.pallas import tpu as pltpu
from jax.experimental.pallas import tpu_sc as plsc

_B = 32
_C = 3
_H = 512
_W = 512
_BT = 32
_BS = _B - _BT
_NC = 2
_NS = 16
_NW = _NC * _NS
_ROWS = 32
_CHUNKS_PER_SLAB = _C * (_H // _ROWS)
_NCHUNK = _BS * _CHUNKS_PER_SLAB
_Q = _NCHUNK // _NW
assert _Q * _NW == _NCHUNK
_LANES = 16
_NBUF = 2
_VECS = _ROWS * _W // _LANES
_TCBUF = 4


def _mesh():
    return plsc.VectorSubcoreMesh(core_axis_name="c", subcore_axis_name="s")


@functools.partial(
    pl.kernel,
    mesh=_mesh(),
    out_type=[
        jax.ShapeDtypeStruct((_NW * _LANES,), jnp.float32),
        jax.ShapeDtypeStruct((_NW * _LANES,), jnp.int32),
    ],
    scratch_types=[
        pltpu.VMEM((_NBUF, _ROWS, _W), jnp.float32),
        pltpu.VMEM((_NBUF, _ROWS, _W), jnp.float32),
        pltpu.VMEM((_NBUF, _ROWS, _W), jnp.int32),
        pltpu.VMEM((_LANES,), jnp.float32),
        pltpu.VMEM((_LANES,), jnp.int32),
        pltpu.SemaphoreType.DMA((_NBUF,)),
    ],
)
def _masked_l1_sc(hat, obs, mask, out_s, out_c, h_v, o_v, m_v, acc_s_v, acc_c_v,
                  sems):
    wid = lax.axis_index("s") * _NC + lax.axis_index("c")
    g0 = wid * _Q

    def chunk_slices(local_idx):
        g = g0 + local_idx
        slab = lax.div(g, _CHUNKS_PER_SLAB)
        rem = lax.rem(g, _CHUNKS_PER_SLAB)
        b_idx = _BT + slab
        ch = lax.shift_right_logical(rem, 4)
        r0 = lax.mul(lax.bitwise_and(rem, 15), _ROWS)
        return b_idx, ch, r0

    def issue(b, local_idx):
        bi, ch, r0 = chunk_slices(local_idx)
        pltpu.async_copy(hat.at[bi, ch, pl.ds(r0, _ROWS), :], h_v.at[b],
                         sems.at[b])
        pltpu.async_copy(obs.at[bi, ch, pl.ds(r0, _ROWS), :], o_v.at[b],
                         sems.at[b])
        pltpu.async_copy(mask.at[bi, ch, pl.ds(r0, _ROWS), :], m_v.at[b],
                         sems.at[b])

    def drain(b, local_idx):
        bi, ch, r0 = chunk_slices(local_idx)
        pltpu.make_async_copy(hat.at[bi, ch, pl.ds(r0, _ROWS), :], h_v.at[b],
                              sems.at[b]).wait()
        pltpu.make_async_copy(obs.at[bi, ch, pl.ds(r0, _ROWS), :], o_v.at[b],
                              sems.at[b]).wait()
        pltpu.make_async_copy(mask.at[bi, ch, pl.ds(r0, _ROWS), :], m_v.at[b],
                              sems.at[b]).wait()

    issue(0, 0)
    issue(1, 1)

    def pair_body(i, carry):
        j = i * _NBUF

        def one(b, carry2):
            s, c = carry2
            jj = j + b
            drain(b, jj)

            def step(k, carry3):
                s3, c3 = carry3
                r = lax.shift_right_logical(k, 5)
                col = lax.mul(lax.bitwise_and(k, 31), _LANES)
                h = h_v[b, r, pl.ds(col, _LANES)]
                o = o_v[b, r, pl.ds(col, _LANES)]
                m = m_v[b, r, pl.ds(col, _LANES)]
                d = jnp.abs(h - o)
                s3 = s3 + d * m.astype(jnp.float32)
                c3 = c3 + m
                return s3, c3

            s, c = lax.fori_loop(0, _VECS, step, (s, c), unroll=8)

            @pl.when(jj + _NBUF < _Q)
            def _():
                issue(b, jj + _NBUF)

            return s, c

        for b in range(_NBUF):
            carry = one(b, carry)
        return carry

    s0 = jnp.zeros((_LANES,), jnp.float32)
    c0 = jnp.zeros((_LANES,), jnp.int32)
    s, c = lax.fori_loop(0, _Q // _NBUF, pair_body, (s0, c0))

    acc_s_v[...] = s
    acc_c_v[...] = c
    pltpu.sync_copy(acc_s_v, out_s.at[pl.ds(wid * _LANES, _LANES)])
    pltpu.sync_copy(acc_c_v, out_c.at[pl.ds(wid * _LANES, _LANES)])


def _tc_body(h_hbm, o_hbm, m_hbm, out_s_ref, out_c_ref,
             h_v, o_v, m_v, acc_s, acc_c, sems):
    def issue(b, bi):
        for ch in range(_C):
            pltpu.make_async_copy(h_hbm.at[bi, ch], h_v.at[b, ch],
                                  sems.at[b]).start()
            pltpu.make_async_copy(o_hbm.at[bi, ch], o_v.at[b, ch],
                                  sems.at[b]).start()
            pltpu.make_async_copy(m_hbm.at[bi, ch], m_v.at[b, ch],
                                  sems.at[b]).start()

    def drain(b, bi):
        for ch in range(_C):
            pltpu.make_async_copy(h_hbm.at[bi, ch], h_v.at[b, ch],
                                  sems.at[b]).wait()
            pltpu.make_async_copy(o_hbm.at[bi, ch], o_v.at[b, ch],
                                  sems.at[b]).wait()
            pltpu.make_async_copy(m_hbm.at[bi, ch], m_v.at[b, ch],
                                  sems.at[b]).wait()

    acc_s[...] = jnp.zeros((_H, _W), jnp.float32)
    acc_c[...] = jnp.zeros((_H, _W), jnp.int32)

    for b0 in range(_TCBUF):
        issue(b0, b0)

    def ring_body(i, _):
        j = i * _TCBUF

        def one(b):
            bi = j + b
            drain(b, bi)
            for ch in range(_C):
                h = h_v[b, ch]
                o = o_v[b, ch]
                m = m_v[b, ch]
                d = jnp.abs(h - o)
                acc_s[...] += d * m.astype(jnp.float32)
                acc_c[...] += m

            @pl.when(bi + _TCBUF < _BT)
            def _():
                issue(b, bi + _TCBUF)

        for b in range(_TCBUF):
            one(b)
        return 0

    lax.fori_loop(0, _BT // _TCBUF, ring_body, 0)

    out_s_ref[0] = jnp.sum(acc_s[...])
    out_c_ref[0] = jnp.sum(acc_c[...])


_tc_part = pl.pallas_call(
    _tc_body,
    in_specs=[
        pl.BlockSpec(memory_space=pltpu.HBM),
        pl.BlockSpec(memory_space=pltpu.HBM),
        pl.BlockSpec(memory_space=pltpu.HBM),
    ],
    out_specs=[
        pl.BlockSpec(memory_space=pltpu.SMEM),
        pl.BlockSpec(memory_space=pltpu.SMEM),
    ],
    out_shape=[
        jax.ShapeDtypeStruct((1,), jnp.float32),
        jax.ShapeDtypeStruct((1,), jnp.int32),
    ],
    scratch_shapes=[
        pltpu.VMEM((_TCBUF, _C, _H, _W), jnp.float32),
        pltpu.VMEM((_TCBUF, _C, _H, _W), jnp.float32),
        pltpu.VMEM((_TCBUF, _C, _H, _W), jnp.int32),
        pltpu.VMEM((_H, _W), jnp.float32),
        pltpu.VMEM((_H, _W), jnp.int32),
        pltpu.SemaphoreType.DMA((_TCBUF,)),
    ],
)


@jax.jit
def kernel(hat, obs, mask):
    if _BS > 0:
        part_s, part_c = _masked_l1_sc(hat, obs, mask)
        tc_s, tc_c = _tc_part(hat, obs, mask)
        total_s = jnp.sum(part_s) + tc_s[0]
        total_c = jnp.sum(part_c) + tc_c[0]
    else:
        tc_s, tc_c = _tc_part(hat, obs, mask)
        total_s = tc_s[0]
        total_c = tc_c[0]
    return total_s / total_c.astype(jnp.float32)


# speedup vs baseline: 1.2575x; 1.0251x over previous
import functools

import jax
import jax.numpy as jnp
from jax import lax
from jax.experimental import pallas as pl
from jax.experimental.pallas import tpu as pltpu
from jax.experimental.pallas import tpu_sc as plsc

_B = 32
_C = 3
_H = 512
_W = 512
_BT = 32
_BS = _B - _BT
_NC = 2
_NS = 16
_NW = _NC * _NS
_ROWS = 32
_CHUNKS_PER_SLAB = _C * (_H // _ROWS)
_NCHUNK = _BS * _CHUNKS_PER_SLAB
_Q = _NCHUNK // _NW
assert _Q * _NW == _NCHUNK
_LANES = 16
_NBUF = 2
_VECS = _ROWS * _W // _LANES
_TCBUF = 4


def _mesh():
    return plsc.VectorSubcoreMesh(core_axis_name="c", subcore_axis_name="s")


@functools.partial(
    pl.kernel,
    mesh=_mesh(),
    out_type=[
        jax.ShapeDtypeStruct((_NW * _LANES,), jnp.float32),
        jax.ShapeDtypeStruct((_NW * _LANES,), jnp.int32),
    ],
    scratch_types=[
        pltpu.VMEM((_NBUF, _ROWS, _W), jnp.float32),
        pltpu.VMEM((_NBUF, _ROWS, _W), jnp.float32),
        pltpu.VMEM((_NBUF, _ROWS, _W), jnp.int32),
        pltpu.VMEM((_LANES,), jnp.float32),
        pltpu.VMEM((_LANES,), jnp.int32),
        pltpu.SemaphoreType.DMA((_NBUF,)),
    ],
)
def _masked_l1_sc(hat, obs, mask, out_s, out_c, h_v, o_v, m_v, acc_s_v, acc_c_v,
                  sems):
    wid = lax.axis_index("s") * _NC + lax.axis_index("c")
    g0 = wid * _Q

    def chunk_slices(local_idx):
        g = g0 + local_idx
        slab = lax.div(g, _CHUNKS_PER_SLAB)
        rem = lax.rem(g, _CHUNKS_PER_SLAB)
        b_idx = _BT + slab
        ch = lax.shift_right_logical(rem, 4)
        r0 = lax.mul(lax.bitwise_and(rem, 15), _ROWS)
        return b_idx, ch, r0

    def issue(b, local_idx):
        bi, ch, r0 = chunk_slices(local_idx)
        pltpu.async_copy(hat.at[bi, ch, pl.ds(r0, _ROWS), :], h_v.at[b],
                         sems.at[b])
        pltpu.async_copy(obs.at[bi, ch, pl.ds(r0, _ROWS), :], o_v.at[b],
                         sems.at[b])
        pltpu.async_copy(mask.at[bi, ch, pl.ds(r0, _ROWS), :], m_v.at[b],
                         sems.at[b])

    def drain(b, local_idx):
        bi, ch, r0 = chunk_slices(local_idx)
        pltpu.make_async_copy(hat.at[bi, ch, pl.ds(r0, _ROWS), :], h_v.at[b],
                              sems.at[b]).wait()
        pltpu.make_async_copy(obs.at[bi, ch, pl.ds(r0, _ROWS), :], o_v.at[b],
                              sems.at[b]).wait()
        pltpu.make_async_copy(mask.at[bi, ch, pl.ds(r0, _ROWS), :], m_v.at[b],
                              sems.at[b]).wait()

    issue(0, 0)
    issue(1, 1)

    def pair_body(i, carry):
        j = i * _NBUF

        def one(b, carry2):
            s, c = carry2
            jj = j + b
            drain(b, jj)

            def step(k, carry3):
                s3, c3 = carry3
                r = lax.shift_right_logical(k, 5)
                col = lax.mul(lax.bitwise_and(k, 31), _LANES)
                h = h_v[b, r, pl.ds(col, _LANES)]
                o = o_v[b, r, pl.ds(col, _LANES)]
                m = m_v[b, r, pl.ds(col, _LANES)]
                d = jnp.abs(h - o)
                s3 = s3 + d * m.astype(jnp.float32)
                c3 = c3 + m
                return s3, c3

            s, c = lax.fori_loop(0, _VECS, step, (s, c), unroll=8)

            @pl.when(jj + _NBUF < _Q)
            def _():
                issue(b, jj + _NBUF)

            return s, c

        for b in range(_NBUF):
            carry = one(b, carry)
        return carry

    s0 = jnp.zeros((_LANES,), jnp.float32)
    c0 = jnp.zeros((_LANES,), jnp.int32)
    s, c = lax.fori_loop(0, _Q // _NBUF, pair_body, (s0, c0))

    acc_s_v[...] = s
    acc_c_v[...] = c
    pltpu.sync_copy(acc_s_v, out_s.at[pl.ds(wid * _LANES, _LANES)])
    pltpu.sync_copy(acc_c_v, out_c.at[pl.ds(wid * _LANES, _LANES)])


def _tc_body(h_hbm, o_hbm, m_hbm, out_s_ref, out_c_ref,
             h_v, o_v, m_v, sems):
    def issue(b, bi):
        for ch in range(_C):
            pltpu.make_async_copy(h_hbm.at[bi, ch], h_v.at[b, ch],
                                  sems.at[b]).start()
            pltpu.make_async_copy(o_hbm.at[bi, ch], o_v.at[b, ch],
                                  sems.at[b]).start()
            pltpu.make_async_copy(m_hbm.at[bi, ch], m_v.at[b, ch],
                                  sems.at[b]).start()

    def drain(b, bi):
        for ch in range(_C):
            pltpu.make_async_copy(h_hbm.at[bi, ch], h_v.at[b, ch],
                                  sems.at[b]).wait()
            pltpu.make_async_copy(o_hbm.at[bi, ch], o_v.at[b, ch],
                                  sems.at[b]).wait()
            pltpu.make_async_copy(m_hbm.at[bi, ch], m_v.at[b, ch],
                                  sems.at[b]).wait()

    for b0 in range(_TCBUF):
        issue(b0, b0)

    def ring_body(i, carry):
        j = i * _TCBUF

        def one(b, carry2):
            a_s, a_c = carry2
            bi = j + b
            drain(b, bi)
            for ch in range(_C):
                h = h_v[b, ch]
                o = o_v[b, ch]
                m = m_v[b, ch]
                d = jnp.abs(h - o) * m.astype(jnp.float32)
                a_s = a_s + jnp.sum(d.reshape(_H // 8, 8, _W), axis=0)
                a_c = a_c + jnp.sum(m.reshape(_H // 8, 8, _W), axis=0)

            @pl.when(bi + _TCBUF < _BT)
            def _():
                issue(b, bi + _TCBUF)

            return a_s, a_c

        for b in range(_TCBUF):
            carry = one(b, carry)
        return carry

    a_s0 = jnp.zeros((8, _W), jnp.float32)
    a_c0 = jnp.zeros((8, _W), jnp.int32)
    a_s, a_c = lax.fori_loop(0, _BT // _TCBUF, ring_body, (a_s0, a_c0))

    out_s_ref[0] = jnp.sum(a_s)
    out_c_ref[0] = jnp.sum(a_c)


_tc_part = pl.pallas_call(
    _tc_body,
    in_specs=[
        pl.BlockSpec(memory_space=pltpu.HBM),
        pl.BlockSpec(memory_space=pltpu.HBM),
        pl.BlockSpec(memory_space=pltpu.HBM),
    ],
    out_specs=[
        pl.BlockSpec(memory_space=pltpu.SMEM),
        pl.BlockSpec(memory_space=pltpu.SMEM),
    ],
    out_shape=[
        jax.ShapeDtypeStruct((1,), jnp.float32),
        jax.ShapeDtypeStruct((1,), jnp.int32),
    ],
    scratch_shapes=[
        pltpu.VMEM((_TCBUF, _C, _H, _W), jnp.float32),
        pltpu.VMEM((_TCBUF, _C, _H, _W), jnp.float32),
        pltpu.VMEM((_TCBUF, _C, _H, _W), jnp.int32),
        pltpu.SemaphoreType.DMA((_TCBUF,)),
    ],
)


@jax.jit
def kernel(hat, obs, mask):
    if _BS > 0:
        part_s, part_c = _masked_l1_sc(hat, obs, mask)
        tc_s, tc_c = _tc_part(hat, obs, mask)
        total_s = jnp.sum(part_s) + tc_s[0]
        total_c = jnp.sum(part_c) + tc_c[0]
    else:
        tc_s, tc_c = _tc_part(hat, obs, mask)
        total_s = tc_s[0]
        total_c = tc_c[0]
    return total_s / total_c.astype(jnp.float32)
